# baseline (speedup 1.0000x reference)
import functools

import jax
import jax.numpy as jnp
from jax import lax
from jax.experimental import pallas as pl
from jax.experimental.pallas import tpu as pltpu
from jax.experimental.pallas import tpu_sc as plsc

_VOCAB = 100000
_D = 300
_DP = 384
_B = 1024
_CTX = 20

_L = 16
_NV = _DP // _L
_NW = 32
_BPW = _B // _NW
_GCHUNK = 4
_NCHUNK = _BPW // _GCHUNK

_RB = 2000


def _renorm_body(t_ref, o_ref):
    x = t_ref[...]
    ss = jnp.sum(x * x, axis=1, keepdims=True)
    scale = jnp.where(ss > 1.0, lax.rsqrt(ss), 1.0)
    o_ref[...] = jnp.concatenate(
        [x * scale, jnp.zeros((_RB, _DP - _D), jnp.float32)], axis=1)


def _renorm_pad(emb_table):
    return pl.pallas_call(
        _renorm_body,
        grid=(_VOCAB // _RB,),
        in_specs=[pl.BlockSpec((_RB, _D), lambda i: (i, 0))],
        out_specs=pl.BlockSpec((_RB, _DP), lambda i: (i, 0)),
        out_shape=jax.ShapeDtypeStruct((_VOCAB, _DP), jnp.float32),
        compiler_params=pltpu.CompilerParams(
            dimension_semantics=("arbitrary",)),
    )(emb_table)


def _sc_pool(idx_flat, table_pad):
    mesh = plsc.VectorSubcoreMesh(core_axis_name="c", subcore_axis_name="s")

    @functools.partial(
        pl.kernel,
        mesh=mesh,
        out_type=jax.ShapeDtypeStruct((_B, _DP), jnp.float32),
        scratch_types=[
            pltpu.VMEM((_BPW * _CTX,), jnp.int32),
            pltpu.VMEM((_GCHUNK * _CTX, _DP), jnp.float32),
            pltpu.VMEM((_BPW, _DP), jnp.float32),
            pltpu.SemaphoreType.DMA,
        ],
    )
    def body(idx_hbm, table_hbm, out_hbm, idx_v, rows_v, outb_v, sem):
        wid = lax.axis_index("s") * 2 + lax.axis_index("c")
        pltpu.sync_copy(idx_hbm.at[pl.ds(wid * (_BPW * _CTX), _BPW * _CTX)],
                        idx_v)

        def chunk_body(c, carry):
            pltpu.async_copy(
                table_hbm.at[idx_v.at[pl.ds(c * (_GCHUNK * _CTX),
                                            _GCHUNK * _CTX)]],
                rows_v, sem).wait()

            def batch_body(b, carry2):
                row0 = b * _CTX

                def row_body(r, accs):
                    row = row0 + r
                    return tuple(
                        accs[j] + rows_v[row, pl.ds(j * _L, _L)]
                        for j in range(_NV))

                accs0 = tuple(jnp.zeros((_L,), jnp.float32)
                              for _ in range(_NV))
                accs = lax.fori_loop(0, _CTX, row_body, accs0)
                gb = c * _GCHUNK + b
                inv = jnp.float32(1.0 / _CTX)
                for j in range(_NV):
                    outb_v[gb, pl.ds(j * _L, _L)] = accs[j] * inv
                return carry2

            lax.fori_loop(0, _GCHUNK, batch_body, 0)
            return carry

        lax.fori_loop(0, _NCHUNK, chunk_body, 0)
        pltpu.sync_copy(outb_v, out_hbm.at[pl.ds(wid * _BPW, _BPW)])

    return body(idx_flat, table_pad)


_VB = 2048
_NVB = pl.cdiv(_VOCAB, _VB)


def _proj_body(h_ref, w_ref, b_ref, o_ref):
    h = h_ref[:, : _D]
    o_ref[...] = lax.dot_general(
        h, w_ref[...],
        dimension_numbers=(((1,), (1,)), ((), ())),
        preferred_element_type=jnp.float32,
    ) + b_ref[...]


def _projection(hidden, lin_w, lin_b2d):
    return pl.pallas_call(
        _proj_body,
        grid=(_NVB,),
        in_specs=[
            pl.BlockSpec((_B, _DP), lambda i: (0, 0)),
            pl.BlockSpec((_VB, _D), lambda i: (i, 0)),
            pl.BlockSpec((1, _VB), lambda i: (0, i)),
        ],
        out_specs=pl.BlockSpec((_B, _VB), lambda i: (0, i)),
        out_shape=jax.ShapeDtypeStruct((_B, _VOCAB), jnp.float32),
        compiler_params=pltpu.CompilerParams(
            dimension_semantics=("arbitrary",)),
    )(hidden, lin_w, lin_b2d)


def kernel(inputs_, emb_table, lin_w, lin_b):
    idx_flat = inputs_.astype(jnp.int32).reshape(-1)
    table_pad = _renorm_pad(emb_table)
    hidden = _sc_pool(idx_flat, table_pad)
    return _projection(hidden, lin_w, lin_b.reshape(1, _VOCAB))
